# baseline (speedup 1.0000x reference)
import dataclasses

import jax
import jax.numpy as jnp
from jax import lax
from jax.experimental import pallas as pl
from jax.experimental.pallas import tpu as pltpu
from jax.experimental.pallas import tpu_sc as plsc

NC = 2
NS = 16
NW = NC * NS
LANES = 16
BW = 128


def _aggregate(mi_hbm, ge_hbm, dr_hbm, idx_hbm, out_hbm,
               idx_v, g0, g1, g2, g3, o_buf, sem):
    wid = lax.axis_index("s") * NC + lax.axis_index("c")
    nl = idx_hbm.shape[0]
    d = mi_hbm.shape[1]
    pltpu.sync_copy(
        idx_hbm.at[pl.ds(0, nl), pl.ds(0, 4), pl.ds(wid * BW, BW)], idx_v)

    @pl.loop(0, nl)
    def _(w):
        c0 = pltpu.async_copy(mi_hbm.at[idx_v.at[w, 0]], g0, sem)
        c1 = pltpu.async_copy(ge_hbm.at[idx_v.at[w, 1]], g1, sem)
        c2 = pltpu.async_copy(ge_hbm.at[idx_v.at[w, 2]], g2, sem)
        c3 = pltpu.async_copy(dr_hbm.at[idx_v.at[w, 3]], g3, sem)
        c0.wait()
        c1.wait()
        c2.wait()
        c3.wait()

        @pl.loop(0, BW)
        def _(r):
            cols = jnp.full((LANES,), r, jnp.int32)
            for c in range(0, d, LANES):
                s = (r, pl.ds(c, LANES))
                val = (g0.at[s][...] + g1.at[s][...]
                       + g2.at[s][...] + g3.at[s][...])
                rows = lax.iota(jnp.int32, LANES) + c
                plsc.store_scatter(o_buf, [rows, cols], val)

        pltpu.sync_copy(
            o_buf, out_hbm.at[w, pl.ds(0, d), pl.ds(wid * BW, BW)])


def _compiler_params():
    cp = pltpu.CompilerParams(use_tc_tiling_on_sc=False)
    if "needs_layout_passes" in pltpu.CompilerParams.__dataclass_fields__:
        cp = dataclasses.replace(cp, needs_layout_passes=False)
    return cp


def kernel(feature_miRNA, feature_gene, feature_drug, mp_ins):
    b, nl, p = mp_ins.shape
    v, d = feature_miRNA.shape
    assert p == 4 and d % LANES == 0 and b % (NW * 8) == 0
    assert b // NW == BW

    idx = jnp.transpose(mp_ins.astype(jnp.int32), (1, 2, 0))

    mesh = plsc.VectorSubcoreMesh(core_axis_name="c", subcore_axis_name="s")
    run = pl.kernel(
        _aggregate,
        out_type=jax.ShapeDtypeStruct((nl, d, b), jnp.float32),
        mesh=mesh,
        scratch_types=[
            pltpu.VMEM((nl, 4, BW), jnp.int32),
            pltpu.VMEM((BW, d), jnp.float32),
            pltpu.VMEM((BW, d), jnp.float32),
            pltpu.VMEM((BW, d), jnp.float32),
            pltpu.VMEM((BW, d), jnp.float32),
            pltpu.VMEM((d, BW), jnp.float32),
            pltpu.SemaphoreType.DMA,
        ],
        compiler_params=_compiler_params(),
    )
    out = run(feature_miRNA, feature_gene, feature_drug, idx)
    return jnp.transpose(out, (2, 0, 1))


# speedup vs baseline: 1.3116x; 1.3116x over previous
import dataclasses

import jax
import jax.numpy as jnp
from jax import lax
from jax.experimental import pallas as pl
from jax.experimental.pallas import tpu as pltpu
from jax.experimental.pallas import tpu_sc as plsc

NC = 2
NS = 16
NW = NC * NS
LANES = 16
BW = 128


def _aggregate(mi_hbm, ge_hbm, dr_hbm, idx_hbm, out_hbm,
               idx_v, g0, g1, g2, g3, o_buf, sem):
    wid = lax.axis_index("s") * NC + lax.axis_index("c")
    nl = idx_hbm.shape[0]
    d = mi_hbm.shape[1]
    pltpu.sync_copy(
        idx_hbm.at[pl.ds(0, nl), pl.ds(0, 4), pl.ds(wid * BW, BW)], idx_v)

    @pl.loop(0, nl)
    def _(w):
        c0 = pltpu.async_copy(mi_hbm.at[idx_v.at[w, 0]], g0, sem)
        c1 = pltpu.async_copy(ge_hbm.at[idx_v.at[w, 1]], g1, sem)
        c2 = pltpu.async_copy(ge_hbm.at[idx_v.at[w, 2]], g2, sem)
        c3 = pltpu.async_copy(dr_hbm.at[idx_v.at[w, 3]], g3, sem)
        c0.wait()
        c1.wait()
        c2.wait()
        c3.wait()

        @pl.loop(0, BW)
        def _(r):
            cols = jnp.full((LANES,), r, jnp.int32)
            for c in range(0, d, LANES):
                s = (r, pl.ds(c, LANES))
                val = (g0.at[s][...] + g1.at[s][...]
                       + g2.at[s][...] + g3.at[s][...])
                rows = lax.iota(jnp.int32, LANES) + c
                plsc.store_scatter(o_buf, [rows, cols], val)

        pltpu.sync_copy(
            o_buf.at[pl.ds(0, d), pl.ds(0, BW)],
            out_hbm.at[w, pl.ds(0, d), pl.ds(wid * BW, BW)])


def _compiler_params():
    cp = pltpu.CompilerParams(use_tc_tiling_on_sc=False)
    if "needs_layout_passes" in pltpu.CompilerParams.__dataclass_fields__:
        cp = dataclasses.replace(cp, needs_layout_passes=False)
    return cp


def kernel(feature_miRNA, feature_gene, feature_drug, mp_ins):
    b, nl, p = mp_ins.shape
    v, d = feature_miRNA.shape
    assert p == 4 and d % LANES == 0 and b % (NW * 8) == 0
    assert b // NW == BW

    idx = jnp.transpose(mp_ins.astype(jnp.int32), (1, 2, 0))

    mesh = plsc.VectorSubcoreMesh(core_axis_name="c", subcore_axis_name="s")
    run = pl.kernel(
        _aggregate,
        out_type=jax.ShapeDtypeStruct((nl, d, b), jnp.float32),
        mesh=mesh,
        scratch_types=[
            pltpu.VMEM((nl, 4, BW), jnp.int32),
            pltpu.VMEM((BW, d), jnp.float32),
            pltpu.VMEM((BW, d), jnp.float32),
            pltpu.VMEM((BW, d), jnp.float32),
            pltpu.VMEM((BW, d), jnp.float32),
            pltpu.VMEM((d, BW + 1), jnp.float32),
            pltpu.SemaphoreType.DMA,
        ],
        compiler_params=_compiler_params(),
    )
    out = run(feature_miRNA, feature_gene, feature_drug, idx)
    return jnp.transpose(out, (2, 0, 1))


# speedup vs baseline: 1.5027x; 1.1458x over previous
import dataclasses

import jax
import jax.numpy as jnp
from jax import lax
from jax.experimental import pallas as pl
from jax.experimental.pallas import tpu as pltpu
from jax.experimental.pallas import tpu_sc as plsc

NC = 2
NS = 16
NW = NC * NS
LANES = 16
BW = 128


def _aggregate(mi_hbm, ge_hbm, dr_hbm, idx_hbm, out_hbm,
               idx_v, g0, g1, g2, g3, sem):
    wid = lax.axis_index("s") * NC + lax.axis_index("c")
    nl = idx_hbm.shape[0]
    d = mi_hbm.shape[1]
    pltpu.sync_copy(
        idx_hbm.at[pl.ds(0, nl), pl.ds(0, 4), pl.ds(wid * BW, BW)], idx_v)

    @pl.loop(0, nl)
    def _(w):
        c0 = pltpu.async_copy(mi_hbm.at[idx_v.at[w, 0]], g0, sem)
        c1 = pltpu.async_copy(ge_hbm.at[idx_v.at[w, 1]], g1, sem)
        c2 = pltpu.async_copy(ge_hbm.at[idx_v.at[w, 2]], g2, sem)
        c3 = pltpu.async_copy(dr_hbm.at[idx_v.at[w, 3]], g3, sem)
        c0.wait()
        c1.wait()
        c2.wait()
        c3.wait()

        @pl.loop(0, BW)
        def _(r):
            for c in range(0, d, LANES):
                s = (r, pl.ds(c, LANES))
                g0.at[s][...] = (g0.at[s][...] + g1.at[s][...]
                                 + g2.at[s][...] + g3.at[s][...])

        pltpu.sync_copy(g0, out_hbm.at[w, pl.ds(wid * BW, BW)])


def _compiler_params():
    cp = pltpu.CompilerParams(use_tc_tiling_on_sc=False)
    if "needs_layout_passes" in pltpu.CompilerParams.__dataclass_fields__:
        cp = dataclasses.replace(cp, needs_layout_passes=False)
    return cp


def kernel(feature_miRNA, feature_gene, feature_drug, mp_ins):
    b, nl, p = mp_ins.shape
    v, d = feature_miRNA.shape
    assert p == 4 and d % LANES == 0 and b == NW * BW

    idx = jnp.transpose(mp_ins.astype(jnp.int32), (1, 2, 0))

    mesh = plsc.VectorSubcoreMesh(core_axis_name="c", subcore_axis_name="s")
    run = pl.kernel(
        _aggregate,
        out_type=jax.ShapeDtypeStruct((nl, b, d), jnp.float32),
        mesh=mesh,
        scratch_types=[
            pltpu.VMEM((nl, 4, BW), jnp.int32),
            pltpu.VMEM((BW, d), jnp.float32),
            pltpu.VMEM((BW, d), jnp.float32),
            pltpu.VMEM((BW, d), jnp.float32),
            pltpu.VMEM((BW, d), jnp.float32),
            pltpu.SemaphoreType.DMA,
        ],
        compiler_params=_compiler_params(),
    )
    out = run(feature_miRNA, feature_gene, feature_drug, idx)
    return jnp.transpose(out, (1, 0, 2))


# speedup vs baseline: 1.7600x; 1.1712x over previous
import dataclasses

import jax
import jax.numpy as jnp
from jax import lax
from jax.experimental import pallas as pl
from jax.experimental.pallas import tpu as pltpu
from jax.experimental.pallas import tpu_sc as plsc

NC = 2
NS = 16
NW = NC * NS
LANES = 16
BW = 128


def _aggregate(mi_hbm, ge_hbm, dr_hbm, idx_hbm, out_hbm,
               idx_v, gA0, gA1, gA2, gA3, gB0, gB1, gB2, gB3, oA, oB,
               semA, semB, osemA, osemB):
    wid = lax.axis_index("s") * NC + lax.axis_index("c")
    nl = idx_hbm.shape[0]
    d = mi_hbm.shape[1]
    sets = {
        "A": ((gA0, gA1, gA2, gA3), oA, semA, osemA),
        "B": ((gB0, gB1, gB2, gB3), oB, semB, osemB),
    }

    pltpu.sync_copy(
        idx_hbm.at[pl.ds(0, nl), pl.ds(0, 4), pl.ds(wid * BW, BW)], idx_v)

    def out_slab(w):
        return out_hbm.at[w, pl.ds(wid * BW, BW)]

    def start_gathers(w, name):
        g, _, sem, _ = sets[name]
        pltpu.async_copy(mi_hbm.at[idx_v.at[w, 0]], g[0], sem)
        pltpu.async_copy(ge_hbm.at[idx_v.at[w, 1]], g[1], sem)
        pltpu.async_copy(ge_hbm.at[idx_v.at[w, 2]], g[2], sem)
        pltpu.async_copy(dr_hbm.at[idx_v.at[w, 3]], g[3], sem)

    def wait_gathers(w, name):
        g, _, sem, _ = sets[name]
        pltpu.make_async_copy(mi_hbm.at[idx_v.at[w, 0]], g[0], sem).wait()
        pltpu.make_async_copy(ge_hbm.at[idx_v.at[w, 1]], g[1], sem).wait()
        pltpu.make_async_copy(ge_hbm.at[idx_v.at[w, 2]], g[2], sem).wait()
        pltpu.make_async_copy(dr_hbm.at[idx_v.at[w, 3]], g[3], sem).wait()

    def wait_out(w, name):
        _, o, _, osem = sets[name]
        pltpu.make_async_copy(o, out_slab(w), osem).wait()

    def half(w, name, other, prefetch):
        g, o, _, osem = sets[name]
        wait_gathers(w, name)
        if prefetch:
            start_gathers(w + 1, other)
        @pl.when(w >= 2)
        def _():
            wait_out(w, name)

        @pl.loop(0, BW)
        def _(r):
            for c in range(0, d, LANES):
                s = (r, pl.ds(c, LANES))
                o.at[s][...] = (g[0].at[s][...] + g[1].at[s][...]
                                + g[2].at[s][...] + g[3].at[s][...])

        pltpu.async_copy(o, out_slab(w), osem)

    start_gathers(0, "A")

    @pl.loop(0, nl // 2)
    def _(i):
        w = 2 * i
        half(w, "A", "B", True)

        @pl.when(i < nl // 2 - 1)
        def _():
            half(w + 1, "B", "A", True)

        @pl.when(i == nl // 2 - 1)
        def _():
            half(w + 1, "B", "A", False)

    wait_out(nl - 2, "A")
    wait_out(nl - 1, "B")


def _compiler_params():
    cp = pltpu.CompilerParams(use_tc_tiling_on_sc=False)
    if "needs_layout_passes" in pltpu.CompilerParams.__dataclass_fields__:
        cp = dataclasses.replace(cp, needs_layout_passes=False)
    return cp


def kernel(feature_miRNA, feature_gene, feature_drug, mp_ins):
    b, nl, p = mp_ins.shape
    v, d = feature_miRNA.shape
    assert p == 4 and d % LANES == 0 and b == NW * BW and nl % 2 == 0

    idx = jnp.transpose(mp_ins.astype(jnp.int32), (1, 2, 0))

    mesh = plsc.VectorSubcoreMesh(core_axis_name="c", subcore_axis_name="s")
    gbuf = pltpu.VMEM((BW, d), jnp.float32)
    run = pl.kernel(
        _aggregate,
        out_type=jax.ShapeDtypeStruct((nl, b, d), jnp.float32),
        mesh=mesh,
        scratch_types=[
            pltpu.VMEM((nl, 4, BW), jnp.int32),
            gbuf, gbuf, gbuf, gbuf, gbuf, gbuf, gbuf, gbuf, gbuf, gbuf,
            pltpu.SemaphoreType.DMA,
            pltpu.SemaphoreType.DMA,
            pltpu.SemaphoreType.DMA,
            pltpu.SemaphoreType.DMA,
        ],
        compiler_params=_compiler_params(),
    )
    out = run(feature_miRNA, feature_gene, feature_drug, idx)
    return jnp.transpose(out, (1, 0, 2))
